# Initial kernel scaffold; baseline (speedup 1.0000x reference)
import jax
import jax.numpy as jnp
from jax.experimental import pallas as pl


def kernel(input, table):
    raise NotImplementedError("write your pallas kernel here")


# speedup vs baseline: 13.5685x; 13.5685x over previous
import functools

import jax
import jax.numpy as jnp
from jax import lax
from jax.experimental import pallas as pl
from jax.experimental.pallas import tpu as pltpu
from jax.experimental.pallas import tpu_sc as plsc

BATCH = 4096
SEQ = 200
DIM = 128
CHUNK = 100
NCHUNK = SEQ // CHUNK
NVREG = DIM // 16


def _tanh(x):
    e = jnp.exp(-2.0 * jnp.abs(x))
    t = (1.0 - e) / (1.0 + e)
    return jnp.where(x < 0, -t, t)


@functools.partial(jax.jit, static_argnums=(2, 3))
def _emb_maxpool(idx, table, nc, ns):
    nw = nc * ns
    bpw = BATCH // nw

    mesh = plsc.VectorSubcoreMesh(core_axis_name="c", subcore_axis_name="s")

    @functools.partial(
        pl.kernel,
        out_type=jax.ShapeDtypeStruct((BATCH, DIM), jnp.float32),
        mesh=mesh,
        scratch_types=[
            pltpu.VMEM((bpw, NCHUNK, CHUNK), jnp.int32),
            pltpu.VMEM((SEQ, DIM), jnp.float32),
            pltpu.VMEM((SEQ, DIM), jnp.float32),
            pltpu.VMEM((bpw, DIM), jnp.float32),
            pltpu.SemaphoreType.DMA,
            pltpu.SemaphoreType.DMA,
        ],
    )
    def k(idx_hbm, table_hbm, out_hbm, idx_v, buf0, buf1, out_v, sem0, sem1):
        wid = lax.axis_index("s") * nc + lax.axis_index("c")
        base = wid * bpw
        pltpu.sync_copy(idx_hbm.at[pl.ds(base, bpw)], idx_v)

        bufs = (buf0, buf1)
        sems = (sem0, sem1)

        def start(r, b):
            for j in range(NCHUNK):
                pltpu.async_copy(
                    table_hbm.at[idx_v.at[r, j]],
                    bufs[b].at[pl.ds(j * CHUNK, CHUNK)],
                    sems[b],
                )

        def wait(r, b):
            for j in range(NCHUNK):
                pltpu.make_async_copy(
                    table_hbm.at[idx_v.at[r, j]],
                    bufs[b].at[pl.ds(j * CHUNK, CHUNK)],
                    sems[b],
                ).wait()

        start(0, 0)
        start(1, 1)

        @pl.loop(0, bpw, step=2)
        def _rows(g):
            for b in range(2):
                r = g + b
                wait(r, b)
                buf = bufs[b]

                def body(s, accs):
                    return tuple(
                        jnp.maximum(a, buf[s, pl.ds(d * 16, 16)])
                        for d, a in enumerate(accs)
                    )

                init = tuple(
                    jnp.full((16,), -jnp.inf, jnp.float32) for _ in range(NVREG)
                )
                accs = lax.fori_loop(0, SEQ, body, init, unroll=8)
                for d in range(NVREG):
                    out_v[r, pl.ds(d * 16, 16)] = _tanh(accs[d])

                nxt = r + 2
                @pl.when(nxt < bpw)
                def _():
                    start(nxt, b)

        pltpu.sync_copy(out_v, out_hbm.at[pl.ds(base, bpw)])

    return k(idx, table)


def kernel(input, table):
    info = plsc.get_sparse_core_info()
    idx = input.reshape(BATCH, NCHUNK, CHUNK)
    return _emb_maxpool(idx, table, info.num_cores, info.num_subcores)
